# baseline (speedup 1.0000x reference)
import functools

import jax
import jax.numpy as jnp
import numpy as np
from jax import lax
from jax.experimental import pallas as pl
from jax.experimental.pallas import tpu as pltpu
from jax.experimental.pallas import tpu_sc as plsc

NG = 512
DIM = 256
NCHUNK = DIM // 16
TILE_R = 128
NWORK = 32
SEG_PER_W = NG // NWORK
NEG = -1e30


def _iota():
    return lax.iota(jnp.int32, 16)


def _vfull(val):
    return jnp.broadcast_to(jnp.float32(val), (16,))


def _perm(v, k):
    return v.at[_iota() ^ k].get(mode="promise_in_bounds")


def _splat(v, j):
    return v.at[_iota() * 0 + j].get(mode="promise_in_bounds")


def _bcast_sum(v):
    for k in (1, 2, 4, 8):
        v = v + _perm(v, k)
    return v


def _bcast_max(v):
    for k in (1, 2, 4, 8):
        v = jnp.maximum(v, _perm(v, k))
    return v


def _sc_pool_body(x_hbm, segoff_hbm, q_hbm, out_hbm, x_tile, x_tile_b, q_all,
                  out_stage, off_a, off_b, carry_g, carry_m, carry_l,
                  sem_xa, sem_xb):
    cid = lax.axis_index("c")
    sid = lax.axis_index("s")
    w = sid * 2 + cid
    seg0 = w * SEG_PER_W
    lanes = lax.iota(jnp.int32, 16)

    pltpu.sync_copy(segoff_hbm.at[pl.ds(seg0, 16)], off_a)
    pltpu.sync_copy(segoff_hbm.at[pl.ds(seg0 + 8, 16)], off_b)
    off_a_v = off_a[...]
    off_b_v = off_b[...]
    offs = [off_a_v[k] for k in range(16)] + [off_b_v[8]]

    def _sel(i):
        acc = offs[0]
        for k in range(1, 17):
            acc = jnp.where(i == k, offs[k], acc)
        return acc

    o_first = offs[0]
    o_last = offs[16]
    wrows = o_last - o_first
    ntile = (wrows + (TILE_R - 1)) // TILE_R

    pltpu.sync_copy(q_hbm.at[pl.ds(seg0 * DIM, SEG_PER_W * DIM)], q_all)

    for z in range(SEG_PER_W * NCHUNK):
        out_stage[pl.ds(z * 16, 16)] = _vfull(0.0)

    for z in range(16 * NCHUNK):
        x_tile[pl.ds(TILE_R * DIM + z * 16, 16)] = _vfull(0.0)
        x_tile_b[pl.ds(TILE_R * DIM + z * 16, 16)] = _vfull(0.0)

    def process_rows(buf, g, a_rel, nr, m_v0, l_v0):
        ngroups = (nr + 15) // 16

        def group_body(i, carry2):
            m_v, l_v = carry2
            rb = a_rel + i * 16
            rem = nr - i * 16

            def row_a(j, s_vec):
                rowoff = (rb + j) * DIM
                pacc = _vfull(0.0)
                for k in range(NCHUNK):
                    pacc = pacc + (buf[pl.ds(rowoff + k * 16, 16)] *
                                   q_all[pl.ds(g * DIM + k * 16, 16)])
                s = _bcast_sum(pacc)
                return jnp.where(lanes == j, s, s_vec)

            s_vec = lax.fori_loop(0, 16, row_a, _vfull(NEG))
            s_vec = jnp.where(lanes < rem, s_vec, _vfull(NEG))

            m_new = jnp.maximum(m_v, _bcast_max(s_vec))
            alpha = jnp.exp(m_v - m_new)
            p_vec = jnp.exp(s_vec - m_new)
            l_new = l_v * alpha + _bcast_sum(p_vec)

            p_s = [_splat(p_vec, jj) for jj in range(16)]
            for k in range(NCHUNK):
                col = k * 16
                acc4 = [out_stage[pl.ds(g * DIM + col, 16)] * alpha,
                        _vfull(0.0), _vfull(0.0), _vfull(0.0)]
                for jj in range(16):
                    acc4[jj % 4] = (acc4[jj % 4] +
                                    p_s[jj] * buf[pl.ds((rb + jj) * DIM + col,
                                                        16)])
                out_stage[pl.ds(g * DIM + col, 16)] = (
                    (acc4[0] + acc4[1]) + (acc4[2] + acc4[3]))
            return m_new, l_new

        return lax.fori_loop(0, ngroups, group_body, (m_v0, l_v0))

    def do_tile(j, mybuf, sem_my, nxtbuf, sem_nxt):
        @pl.when(j + 1 < ntile)
        def _():
            pltpu.async_copy(
                x_hbm.at[pl.ds((o_first + (j + 1) * TILE_R) * DIM,
                               TILE_R * DIM)],
                nxtbuf.at[pl.ds(0, TILE_R * DIM)], sem_nxt)

        pltpu.make_async_copy(
            x_hbm.at[pl.ds((o_first + j * TILE_R) * DIM, TILE_R * DIM)],
            mybuf.at[pl.ds(0, TILE_R * DIM)], sem_my).wait()

        t0 = o_first + j * TILE_R
        t1 = jnp.minimum(t0 + TILE_R, o_last)

        shifted = off_a_v.at[jnp.minimum(lanes + 1, 15)].get(
            mode="promise_in_bounds")
        ends_v = jnp.where(lanes == 15, offs[16], shifted)
        cnt_v = _bcast_sum(jnp.where(ends_v <= t1, 1, 0))
        cnt = cnt_v[0]
        g0 = carry_g[...][0]
        steps = cnt - g0 + jnp.where(cnt < SEG_PER_W, 1, 0)

        def step_body(k, carry):
            m_v, l_v = carry
            g = g0 + k
            og = _sel(g)
            og1 = _sel(g + 1)
            a = jnp.maximum(og, t0)
            b = jnp.minimum(og1, t1)
            nr = jnp.maximum(b - a, 0)
            m_v, l_v = process_rows(mybuf, g, a - t0, nr, m_v, l_v)
            fin = og1 <= t1
            scale = jnp.where(fin, 1.0 / (l_v + 1e-8), _vfull(1.0))
            for k2 in range(NCHUNK):
                out_stage[pl.ds(g * DIM + k2 * 16, 16)] = (
                    out_stage[pl.ds(g * DIM + k2 * 16, 16)] * scale)
            m_v = jnp.where(fin, _vfull(0.0), m_v)
            l_v = jnp.where(fin, _vfull(0.0), l_v)
            return m_v, l_v

        m_v, l_v = lax.fori_loop(0, steps, step_body,
                                 (carry_m[...], carry_l[...]))
        carry_g[...] = jnp.broadcast_to(cnt, (16,))
        carry_m[...] = m_v
        carry_l[...] = l_v

    carry_g[...] = _iota() * 0
    carry_m[...] = _vfull(0.0)
    carry_l[...] = _vfull(0.0)

    @pl.when(ntile > 0)
    def _():
        pltpu.async_copy(x_hbm.at[pl.ds(o_first * DIM, TILE_R * DIM)],
                         x_tile.at[pl.ds(0, TILE_R * DIM)], sem_xa)

    def tile_loop(j, _c):
        @pl.when(j % 2 == 0)
        def _():
            do_tile(j, x_tile, sem_xa, x_tile_b, sem_xb)

        @pl.when(j % 2 == 1)
        def _():
            do_tile(j, x_tile_b, sem_xb, x_tile, sem_xa)
        return 0

    lax.fori_loop(0, ntile, tile_loop, 0)
    pltpu.sync_copy(out_stage, out_hbm.at[pl.ds(seg0 * DIM, SEG_PER_W * DIM)])


def _make_sc_pool():
    mesh = plsc.VectorSubcoreMesh(core_axis_name="c", subcore_axis_name="s")
    return functools.partial(
        pl.kernel,
        mesh=mesh,
        out_type=jax.ShapeDtypeStruct((NG * DIM,), jnp.float32),
        scratch_types=[
            pltpu.VMEM(((TILE_R + 16) * DIM,), jnp.float32),
            pltpu.VMEM(((TILE_R + 16) * DIM,), jnp.float32),
            pltpu.VMEM((SEG_PER_W * DIM,), jnp.float32),
            pltpu.VMEM((SEG_PER_W * DIM,), jnp.float32),
            pltpu.VMEM((16,), jnp.int32),
            pltpu.VMEM((16,), jnp.int32),
            pltpu.VMEM((16,), jnp.int32),
            pltpu.VMEM((16,), jnp.float32),
            pltpu.VMEM((16,), jnp.float32),
            pltpu.SemaphoreType.DMA,
            pltpu.SemaphoreType.DMA,
        ],
    )(_sc_pool_body)


def _lstm_tc(r_ref, h_ref, c_ref, wih_ref, whh_ref, b_ref, h_out, c_out):
    r = r_ref[...]
    h = h_ref[...]
    gates = (lax.dot_general(r, wih_ref[...], (((1,), (1,)), ((), ())),
                             preferred_element_type=jnp.float32)
             + lax.dot_general(h, whh_ref[...], (((1,), (1,)), ((), ())),
                               preferred_element_type=jnp.float32)
             + b_ref[...][None, :])
    i = jax.nn.sigmoid(gates[:, :DIM])
    f = jax.nn.sigmoid(gates[:, DIM:2 * DIM])
    g = jnp.tanh(gates[:, 2 * DIM:3 * DIM])
    o = jax.nn.sigmoid(gates[:, 3 * DIM:])
    c_new = f * c_ref[...] + i * g
    h_out[...] = o * jnp.tanh(c_new)
    c_out[...] = c_new


def _lstm_call(r, h, c, W_ih, W_hh, b):
    return pl.pallas_call(
        _lstm_tc,
        out_shape=[jax.ShapeDtypeStruct((NG, DIM), jnp.float32),
                   jax.ShapeDtypeStruct((NG, DIM), jnp.float32)],
    )(r, h, c, W_ih, W_hh, b)


def kernel(x, batch, W_ih, W_hh, b_ih, b_hh):
    n = x.shape[0]
    npad = n + TILE_R + 48
    xp = jnp.pad(x, ((0, npad - n), (0, 0))).reshape(-1)
    seg_off = jnp.searchsorted(batch, jnp.arange(NG + 1, dtype=jnp.int32),
                               side="left").astype(jnp.int32)
    seg_off_p = jnp.pad(seg_off, (0, 527 - NG))
    b = (b_ih + b_hh).astype(jnp.float32)

    sc_pool = _make_sc_pool()
    h = jnp.zeros((NG, DIM), jnp.float32)
    c = jnp.zeros((NG, DIM), jnp.float32)
    readout = None
    for _ in range(6):
        readout = sc_pool(xp, seg_off_p, h.reshape(-1)).reshape(NG, DIM)
        h, c = _lstm_call(readout, h, c, W_ih, W_hh, b)
    return jnp.concatenate([h, readout], axis=-1)
